# baseline (speedup 1.0000x reference)
import functools

import jax
import jax.numpy as jnp
from jax import lax
from jax.experimental import pallas as pl
from jax.experimental.pallas import tpu as pltpu
from jax.experimental.pallas import tpu_sc as plsc

N_BUCKETS = 512
SLOTS = 32
D = 128
NCH = D // 16
TAU = 0.1
L = 16


def _rsqrt16(x):
    i = lax.bitcast_convert_type(x, jnp.int32)
    i = jnp.int32(0x5F3759DF) - (i >> 1)
    y = lax.bitcast_convert_type(i, jnp.float32)
    for _ in range(3):
        y = y * (1.5 - 0.5 * x * y * y)
    return y


def _bcast_lane(v, lane):
    idx = jnp.full((L,), lane, jnp.int32)
    return v.at[idx].get(mode="promise_in_bounds")


def _round_bf16(v):
    i = lax.bitcast_convert_type(v, jnp.int32)
    i = i + jnp.int32(0x7FFF) + ((i >> 16) & 1)
    i = i & jnp.int32(-65536)
    return lax.bitcast_convert_type(i, jnp.float32)


def _sum_all(v, lanes):
    for sh in (8, 4, 2, 1):
        idx = lanes ^ sh
        v = v + v.at[idx].get(mode="promise_in_bounds")
    return v


def _max_all(v, lanes):
    for sh in (8, 4, 2, 1):
        idx = lanes ^ sh
        v = jnp.maximum(v, v.at[idx].get(mode="promise_in_bounds"))
    return v


def _unpack2(w):
    a = lax.bitcast_convert_type(w << 16, jnp.float32)
    b = lax.bitcast_convert_type(w & jnp.int32(-65536), jnp.float32)
    return a, b


def _make_sc_call(num_queries, qpw, chunk):
    mesh = plsc.VectorSubcoreMesh(core_axis_name="c", subcore_axis_name="s",
                                  num_cores=2, num_subcores=16)
    nc = 2
    grids = qpw // chunk
    assert grids % 2 == 0 and chunk == 8

    @functools.partial(
        pl.kernel,
        out_type=(
            jax.ShapeDtypeStruct((num_queries, D), jnp.float32),
            jax.ShapeDtypeStruct((num_queries,), jnp.float32),
            jax.ShapeDtypeStruct((num_queries,), jnp.int32),
        ),
        mesh=mesh,
        scratch_types=dict(
            tid_v=pltpu.VMEM((qpw,), jnp.int32),
            bkt_v=pltpu.VMEM((qpw,), jnp.int32),
            ktbuf=pltpu.VMEM((2, chunk, D * SLOTS // 2), jnp.int32),
            vbuf=pltpu.VMEM((2, chunk, SLOTS * D // 2), jnp.int32),
            stbuf=pltpu.VMEM((2, chunk, D), jnp.int32),
            cbuf=pltpu.VMEM((2, chunk, D), jnp.float32),
            qbuf=pltpu.VMEM((2, chunk, D), jnp.float32),
            uqbuf=pltpu.VMEM((NCH, L), jnp.float32),
            ovbuf=pltpu.VMEM((chunk, D), jnp.float32),
            msbuf=pltpu.VMEM((qpw,), jnp.float32),
            sems=pltpu.SemaphoreType.DMA((2, 5)),
        ),
    )
    def sc_call(qf, kt, vals, stids, cents, tidsf, val_out, ms_out, bk_out,
                tid_v, bkt_v, ktbuf, vbuf, stbuf, cbuf, qbuf, uqbuf,
                ovbuf, msbuf, sems):
        wid = lax.axis_index("s") * nc + lax.axis_index("c")
        base = wid * qpw

        pltpu.sync_copy(tidsf.at[pl.ds(base, qpw)], tid_v)

        def bkt_body(i, _):
            t16 = tid_v[pl.ds(i * L, L)]
            bkt_v[pl.ds(i * L, L)] = t16 & jnp.int32(N_BUCKETS - 1)
            return 0

        lax.fori_loop(0, qpw // L, bkt_body, 0)
        pltpu.sync_copy(bkt_v, bk_out.at[pl.ds(base, qpw)])

        zero16 = jnp.zeros((L,), jnp.float32)
        lanes = lax.iota(jnp.int32, L)

        def fire(g, s):
            idx = bkt_v.at[pl.ds(g * chunk, chunk)]
            pltpu.async_copy(kt.at[idx], ktbuf.at[s], sems.at[s, 0])
            pltpu.async_copy(vals.at[idx], vbuf.at[s], sems.at[s, 1])
            pltpu.async_copy(stids.at[idx], stbuf.at[s], sems.at[s, 2])
            pltpu.async_copy(cents.at[idx], cbuf.at[s], sems.at[s, 3])
            pltpu.async_copy(qf.at[pl.ds(base + g * chunk, chunk)],
                             qbuf.at[s], sems.at[s, 4])

        def wait(g, s):
            idx = bkt_v.at[pl.ds(g * chunk, chunk)]
            pltpu.make_async_copy(kt.at[idx], ktbuf.at[s], sems.at[s, 0]).wait()
            pltpu.make_async_copy(vals.at[idx], vbuf.at[s], sems.at[s, 1]).wait()
            pltpu.make_async_copy(stids.at[idx], stbuf.at[s], sems.at[s, 2]).wait()
            pltpu.make_async_copy(cents.at[idx], cbuf.at[s], sems.at[s, 3]).wait()
            pltpu.make_async_copy(qf.at[pl.ds(base + g * chunk, chunk)],
                                  qbuf.at[s], sems.at[s, 4]).wait()

        def compute(g, s, lane0, tchunk, msv):
            row0 = g * chunk
            for qi in range(chunk):
                qs = [qbuf[s, qi, pl.ds(c * L, L)] for c in range(NCH)]
                nsq = zero16
                for q_c in qs:
                    nsq = nsq + q_c * q_c
                nsq = jnp.maximum(_sum_all(nsq, lanes), 1e-24)
                rq = _rsqrt16(nsq)
                ts = [qs[c] * rq + cbuf[s, qi, pl.ds(c * L, L)]
                      for c in range(NCH)]
                tsq = zero16
                for t_c in ts:
                    tsq = tsq + t_c * t_c
                tsq = jnp.maximum(_sum_all(tsq, lanes), 1e-24)
                rt = _rsqrt16(tsq)
                for c in range(NCH):
                    uqbuf[c, :] = _round_bf16(ts[c] * rt)

                def score_body(c, carry):
                    a0, a1 = carry
                    uq_c = uqbuf[c, :]
                    for j in range(L):
                        u = _bcast_lane(uq_c, j)
                        dd = c * L + j
                        k0, k1 = _unpack2(
                            ktbuf[s, qi, pl.ds(dd * L, L)])
                        a0 = a0 + u * k0
                        a1 = a1 + u * k1
                    return a0, a1

                s0, s1 = lax.fori_loop(0, NCH, score_body, (zero16, zero16))

                tidv = _bcast_lane(tchunk, lane0 + qi)
                mf0 = 1.0 - jnp.minimum(
                    jnp.abs(stbuf[s, qi, pl.ds(0, L)] - tidv), 1
                ).astype(jnp.float32)
                mf1 = 1.0 - jnp.minimum(
                    jnp.abs(stbuf[s, qi, pl.ds(L, L)] - tidv), 1
                ).astype(jnp.float32)
                msum = _sum_all(mf0 + mf1, lanes)
                hasf = jnp.minimum(msum, 1.0)

                smax = _max_all(jnp.maximum(s0, s1), lanes)
                e0 = jnp.exp((s0 - smax) * (1.0 / TAU))
                e1 = jnp.exp((s1 - smax) * (1.0 / TAU))
                zinv = 1.0 / _sum_all(e0 + e1, lanes)
                hinv = 1.0 / (msum + 1e-9)
                p0 = _round_bf16(
                    hasf * (mf0 * hinv) + (1.0 - hasf) * (e0 * zinv))
                p1 = _round_bf16(
                    hasf * (mf1 * hinv) + (1.0 - hasf) * (e1 * zinv))
                ms_q = hasf * 10.0 + (1.0 - hasf) * smax
                lm = jnp.minimum(
                    jnp.abs(lanes - (lane0 + qi)), 1).astype(jnp.float32)
                msv = msv * lm + ms_q * (1.0 - lm)

                def val_body(so, accs):
                    sl = so & (L - 1)
                    svec = jnp.full((L,), so, jnp.int32)
                    hi = jnp.minimum(jnp.maximum(svec - (L - 1), 0),
                                     1).astype(jnp.float32)
                    b = (1.0 - hi) * _bcast_lane(p0, sl) \
                        + hi * _bcast_lane(p1, sl)
                    out = []
                    for c in range(NCH // 2):
                        va, vb = _unpack2(
                            vbuf[s, qi, pl.ds(so * (D // 2) + c * L, L)])
                        out.append(accs[2 * c] + b * va)
                        out.append(accs[2 * c + 1] + b * vb)
                    return tuple(out)

                accs = lax.fori_loop(0, SLOTS, val_body, (zero16,) * NCH,
                                     unroll=4)
                for c in range(NCH):
                    ovbuf[qi, pl.ds(c * L, L)] = accs[c]

            pltpu.sync_copy(ovbuf, val_out.at[pl.ds(base + row0, chunk)])
            return msv

        fire(0, 0)

        def pair_body(h, _):
            g0 = 2 * h
            tchunk = tid_v[pl.ds(h * L, L)]
            fire(g0 + 1, 1)
            wait(g0, 0)
            msv = compute(g0, 0, 0, tchunk, zero16)

            @pl.when(h + 1 < grids // 2)
            def _():
                fire(g0 + 2, 0)

            wait(g0 + 1, 1)
            msv = compute(g0 + 1, 1, chunk, tchunk, msv)
            msbuf[pl.ds(g0 * chunk, L)] = msv
            return 0

        lax.fori_loop(0, grids // 2, pair_body, 0)
        pltpu.sync_copy(msbuf, ms_out.at[pl.ds(base, qpw)])

    return sc_call


def kernel(query_emb, slot_values, slot_keys, tids, centroid_codebook,
           slot_tids):
    B, T, d = query_emb.shape
    nq = B * T
    nw = 32
    qpw = nq // nw

    qf = query_emb.reshape(nq, d)
    ktf = jnp.transpose(slot_keys[0].reshape(N_BUCKETS, SLOTS, d), (0, 2, 1))
    kt = lax.bitcast_convert_type(
        ktf.astype(jnp.bfloat16)
        .reshape(N_BUCKETS, d, 2, L)
        .transpose(0, 1, 3, 2)
        .reshape(N_BUCKETS, d * SLOTS // 2, 2), jnp.int32)
    vals = lax.bitcast_convert_type(
        slot_values.reshape(N_BUCKETS, SLOTS, d).astype(jnp.bfloat16)
        .reshape(N_BUCKETS, SLOTS, NCH // 2, 2, L)
        .transpose(0, 1, 2, 4, 3)
        .reshape(N_BUCKETS, SLOTS * d // 2, 2), jnp.int32)
    stids = jnp.pad(slot_tids[0].reshape(N_BUCKETS, SLOTS).astype(jnp.int32),
                    ((0, 0), (0, d - SLOTS)), constant_values=-1)
    cents = centroid_codebook
    tidsf = tids.reshape(nq).astype(jnp.int32)

    sc_call = _make_sc_call(nq, qpw, chunk=8)
    val, ms, bk = sc_call(qf, kt, vals, stids, cents, tidsf)
    return val.reshape(B, T, d), ms.reshape(B, T), bk.reshape(B, T)


# speedup vs baseline: 1.0793x; 1.0793x over previous
import functools

import jax
import jax.numpy as jnp
from jax import lax
from jax.experimental import pallas as pl
from jax.experimental.pallas import tpu as pltpu
from jax.experimental.pallas import tpu_sc as plsc

N_BUCKETS = 512
SLOTS = 32
D = 128
NCH = D // 16
TAU = 0.1
L = 16


def _rsqrt16(x):
    i = lax.bitcast_convert_type(x, jnp.int32)
    i = jnp.int32(0x5F3759DF) - (i >> 1)
    y = lax.bitcast_convert_type(i, jnp.float32)
    for _ in range(3):
        y = y * (1.5 - 0.5 * x * y * y)
    return y


def _bcast_lane(v, lane):
    idx = jnp.full((L,), lane, jnp.int32)
    return v.at[idx].get(mode="promise_in_bounds")


def _round_bf16(v):
    i = lax.bitcast_convert_type(v, jnp.int32)
    i = i + jnp.int32(0x7FFF) + ((i >> 16) & 1)
    i = i & jnp.int32(-65536)
    return lax.bitcast_convert_type(i, jnp.float32)


def _sum_all(v, lanes):
    for sh in (8, 4, 2, 1):
        idx = lanes ^ sh
        v = v + v.at[idx].get(mode="promise_in_bounds")
    return v


def _max_all(v, lanes):
    for sh in (8, 4, 2, 1):
        idx = lanes ^ sh
        v = jnp.maximum(v, v.at[idx].get(mode="promise_in_bounds"))
    return v


def _unpack2(w):
    a = lax.bitcast_convert_type(w << 16, jnp.float32)
    b = lax.bitcast_convert_type(w & jnp.int32(-65536), jnp.float32)
    return a, b


def _make_sc_call(num_queries, qpw, chunk):
    mesh = plsc.VectorSubcoreMesh(core_axis_name="c", subcore_axis_name="s",
                                  num_cores=2, num_subcores=16)
    nc = 2
    grids = qpw // chunk
    assert grids % 2 == 0 and chunk == 8

    @functools.partial(
        pl.kernel,
        out_type=(
            jax.ShapeDtypeStruct((num_queries, D), jnp.float32),
            jax.ShapeDtypeStruct((num_queries,), jnp.float32),
            jax.ShapeDtypeStruct((num_queries,), jnp.int32),
        ),
        mesh=mesh,
        scratch_types=dict(
            tid_v=pltpu.VMEM((qpw,), jnp.int32),
            bkt_v=pltpu.VMEM((qpw,), jnp.int32),
            ktbuf=pltpu.VMEM((2, chunk, D * SLOTS // 2), jnp.int32),
            vbuf=pltpu.VMEM((2, chunk, SLOTS * D // 2), jnp.int32),
            stbuf=pltpu.VMEM((2, chunk, D), jnp.int32),
            cbuf=pltpu.VMEM((2, chunk, D), jnp.float32),
            qbuf=pltpu.VMEM((2, chunk, D), jnp.float32),
            uqbuf=pltpu.VMEM((NCH, L), jnp.float32),
            ovbuf=pltpu.VMEM((chunk, D), jnp.float32),
            msbuf=pltpu.VMEM((qpw,), jnp.float32),
            sems=pltpu.SemaphoreType.DMA((2, 5)),
        ),
    )
    def sc_call(qf, kt, vals, stids, cents, tidsf, val_out, ms_out, bk_out,
                tid_v, bkt_v, ktbuf, vbuf, stbuf, cbuf, qbuf, uqbuf,
                ovbuf, msbuf, sems):
        wid = lax.axis_index("s") * nc + lax.axis_index("c")
        base = wid * qpw

        pltpu.sync_copy(tidsf.at[pl.ds(base, qpw)], tid_v)

        def bkt_body(i, _):
            t16 = tid_v[pl.ds(i * L, L)]
            bkt_v[pl.ds(i * L, L)] = t16 & jnp.int32(N_BUCKETS - 1)
            return 0

        lax.fori_loop(0, qpw // L, bkt_body, 0)
        pltpu.sync_copy(bkt_v, bk_out.at[pl.ds(base, qpw)])

        zero16 = jnp.zeros((L,), jnp.float32)
        lanes = lax.iota(jnp.int32, L)

        def fire(g, s):
            idx = bkt_v.at[pl.ds(g * chunk, chunk)]
            pltpu.async_copy(kt.at[idx], ktbuf.at[s], sems.at[s, 0])
            pltpu.async_copy(vals.at[idx], vbuf.at[s], sems.at[s, 1])
            pltpu.async_copy(stids.at[idx], stbuf.at[s], sems.at[s, 2])
            pltpu.async_copy(cents.at[idx], cbuf.at[s], sems.at[s, 3])
            pltpu.async_copy(qf.at[pl.ds(base + g * chunk, chunk)],
                             qbuf.at[s], sems.at[s, 4])

        def wait(g, s):
            idx = bkt_v.at[pl.ds(g * chunk, chunk)]
            pltpu.make_async_copy(kt.at[idx], ktbuf.at[s], sems.at[s, 0]).wait()
            pltpu.make_async_copy(vals.at[idx], vbuf.at[s], sems.at[s, 1]).wait()
            pltpu.make_async_copy(stids.at[idx], stbuf.at[s], sems.at[s, 2]).wait()
            pltpu.make_async_copy(cents.at[idx], cbuf.at[s], sems.at[s, 3]).wait()
            pltpu.make_async_copy(qf.at[pl.ds(base + g * chunk, chunk)],
                                  qbuf.at[s], sems.at[s, 4]).wait()

        def compute(g, s, lane0, tchunk, msv):
            row0 = g * chunk
            for qi in range(chunk):
                qs = [qbuf[s, qi, pl.ds(c * L, L)] for c in range(NCH)]
                nsq = zero16
                for q_c in qs:
                    nsq = nsq + q_c * q_c
                nsq = jnp.maximum(_sum_all(nsq, lanes), 1e-24)
                rq = _rsqrt16(nsq)
                ts = [qs[c] * rq + cbuf[s, qi, pl.ds(c * L, L)]
                      for c in range(NCH)]
                tsq = zero16
                for t_c in ts:
                    tsq = tsq + t_c * t_c
                tsq = jnp.maximum(_sum_all(tsq, lanes), 1e-24)
                rt = _rsqrt16(tsq)
                for c in range(NCH):
                    uqbuf[c, :] = _round_bf16(ts[c] * rt)

                def score_body(c, carry):
                    acc = list(carry)
                    uq_c = uqbuf[c, :]
                    for j in range(L):
                        u = _bcast_lane(uq_c, j)
                        dd = c * L + j
                        k0, k1 = _unpack2(
                            ktbuf[s, qi, pl.ds(dd * L, L)])
                        acc[j & 3] = acc[j & 3] + u * k0
                        acc[4 + (j & 3)] = acc[4 + (j & 3)] + u * k1
                    return tuple(acc)

                sa = lax.fori_loop(0, NCH, score_body, (zero16,) * 8)
                s0 = (sa[0] + sa[1]) + (sa[2] + sa[3])
                s1 = (sa[4] + sa[5]) + (sa[6] + sa[7])

                tidv = _bcast_lane(tchunk, lane0 + qi)
                mf0 = 1.0 - jnp.minimum(
                    jnp.abs(stbuf[s, qi, pl.ds(0, L)] - tidv), 1
                ).astype(jnp.float32)
                mf1 = 1.0 - jnp.minimum(
                    jnp.abs(stbuf[s, qi, pl.ds(L, L)] - tidv), 1
                ).astype(jnp.float32)
                msum = _sum_all(mf0 + mf1, lanes)
                hasf = jnp.minimum(msum, 1.0)

                smax = _max_all(jnp.maximum(s0, s1), lanes)
                e0 = jnp.exp((s0 - smax) * (1.0 / TAU))
                e1 = jnp.exp((s1 - smax) * (1.0 / TAU))
                zinv = 1.0 / _sum_all(e0 + e1, lanes)
                hinv = 1.0 / (msum + 1e-9)
                p0 = _round_bf16(
                    hasf * (mf0 * hinv) + (1.0 - hasf) * (e0 * zinv))
                p1 = _round_bf16(
                    hasf * (mf1 * hinv) + (1.0 - hasf) * (e1 * zinv))
                ms_q = hasf * 10.0 + (1.0 - hasf) * smax
                lm = jnp.minimum(
                    jnp.abs(lanes - (lane0 + qi)), 1).astype(jnp.float32)
                msv = msv * lm + ms_q * (1.0 - lm)

                def val_body(so, accs):
                    sl = so & (L - 1)
                    svec = jnp.full((L,), so, jnp.int32)
                    hi = jnp.minimum(jnp.maximum(svec - (L - 1), 0),
                                     1).astype(jnp.float32)
                    b = (1.0 - hi) * _bcast_lane(p0, sl) \
                        + hi * _bcast_lane(p1, sl)
                    out = []
                    for c in range(NCH // 2):
                        va, vb = _unpack2(
                            vbuf[s, qi, pl.ds(so * (D // 2) + c * L, L)])
                        out.append(accs[2 * c] + b * va)
                        out.append(accs[2 * c + 1] + b * vb)
                    return tuple(out)

                accs = lax.fori_loop(0, SLOTS, val_body, (zero16,) * NCH,
                                     unroll=2)
                for c in range(NCH):
                    ovbuf[qi, pl.ds(c * L, L)] = accs[c]

            pltpu.sync_copy(ovbuf, val_out.at[pl.ds(base + row0, chunk)])
            return msv

        fire(0, 0)

        def pair_body(h, _):
            g0 = 2 * h
            tchunk = tid_v[pl.ds(h * L, L)]
            fire(g0 + 1, 1)
            wait(g0, 0)
            msv = compute(g0, 0, 0, tchunk, zero16)

            @pl.when(h + 1 < grids // 2)
            def _():
                fire(g0 + 2, 0)

            wait(g0 + 1, 1)
            msv = compute(g0 + 1, 1, chunk, tchunk, msv)
            msbuf[pl.ds(g0 * chunk, L)] = msv
            return 0

        lax.fori_loop(0, grids // 2, pair_body, 0)
        pltpu.sync_copy(msbuf, ms_out.at[pl.ds(base, qpw)])

    return sc_call


def kernel(query_emb, slot_values, slot_keys, tids, centroid_codebook,
           slot_tids):
    B, T, d = query_emb.shape
    nq = B * T
    nw = 32
    qpw = nq // nw

    qf = query_emb.reshape(nq, d)
    ktf = jnp.transpose(slot_keys[0].reshape(N_BUCKETS, SLOTS, d), (0, 2, 1))
    kt = lax.bitcast_convert_type(
        ktf.astype(jnp.bfloat16)
        .reshape(N_BUCKETS, d, 2, L)
        .transpose(0, 1, 3, 2)
        .reshape(N_BUCKETS, d * SLOTS // 2, 2), jnp.int32)
    vals = lax.bitcast_convert_type(
        slot_values.reshape(N_BUCKETS, SLOTS, d).astype(jnp.bfloat16)
        .reshape(N_BUCKETS, SLOTS, NCH // 2, 2, L)
        .transpose(0, 1, 2, 4, 3)
        .reshape(N_BUCKETS, SLOTS * d // 2, 2), jnp.int32)
    stids = jnp.pad(slot_tids[0].reshape(N_BUCKETS, SLOTS).astype(jnp.int32),
                    ((0, 0), (0, d - SLOTS)), constant_values=-1)
    cents = centroid_codebook
    tidsf = tids.reshape(nq).astype(jnp.int32)

    sc_call = _make_sc_call(nq, qpw, chunk=8)
    val, ms, bk = sc_call(qf, kt, vals, stids, cents, tidsf)
    return val.reshape(B, T, d), ms.reshape(B, T), bk.reshape(B, T)


# speedup vs baseline: 1.3160x; 1.2193x over previous
import functools

import jax
import jax.numpy as jnp
from jax import lax
from jax.experimental import pallas as pl
from jax.experimental.pallas import tpu as pltpu
from jax.experimental.pallas import tpu_sc as plsc

N_BUCKETS = 512
SLOTS = 32
D = 128
NCH = D // 16
TAU = 0.1
L = 16


def _rsqrt16(x):
    i = lax.bitcast_convert_type(x, jnp.int32)
    i = jnp.int32(0x5F3759DF) - (i >> 1)
    y = lax.bitcast_convert_type(i, jnp.float32)
    for _ in range(3):
        y = y * (1.5 - 0.5 * x * y * y)
    return y


def _bcast_lane(v, lane):
    idx = jnp.full((L,), lane, jnp.int32)
    return v.at[idx].get(mode="promise_in_bounds")


def _round_bf16(v):
    i = lax.bitcast_convert_type(v, jnp.int32)
    i = i + jnp.int32(0x7FFF) + ((i >> 16) & 1)
    i = i & jnp.int32(-65536)
    return lax.bitcast_convert_type(i, jnp.float32)


def _sum_all(v, lanes):
    for sh in (8, 4, 2, 1):
        idx = lanes ^ sh
        v = v + v.at[idx].get(mode="promise_in_bounds")
    return v


def _max_all(v, lanes):
    for sh in (8, 4, 2, 1):
        idx = lanes ^ sh
        v = jnp.maximum(v, v.at[idx].get(mode="promise_in_bounds"))
    return v


def _unpack2(w):
    a = lax.bitcast_convert_type(w << 16, jnp.float32)
    b = lax.bitcast_convert_type(w & jnp.int32(-65536), jnp.float32)
    return a, b


def _make_sc_call(num_queries, qpw, chunk):
    mesh = plsc.VectorSubcoreMesh(core_axis_name="c", subcore_axis_name="s",
                                  num_cores=2, num_subcores=16)
    nc = 2
    grids = qpw // chunk
    assert grids % 2 == 0 and chunk == 8

    @functools.partial(
        pl.kernel,
        out_type=(
            jax.ShapeDtypeStruct((num_queries, D), jnp.float32),
            jax.ShapeDtypeStruct((num_queries,), jnp.float32),
            jax.ShapeDtypeStruct((num_queries,), jnp.int32),
        ),
        mesh=mesh,
        scratch_types=dict(
            tid_v=pltpu.VMEM((qpw,), jnp.int32),
            bkt_v=pltpu.VMEM((qpw,), jnp.int32),
            ktbuf=pltpu.VMEM((2, chunk, D * SLOTS // 2), jnp.int32),
            vbuf=pltpu.VMEM((2, chunk, SLOTS * D // 2), jnp.int32),
            stbuf=pltpu.VMEM((2, chunk, D), jnp.int32),
            cbuf=pltpu.VMEM((2, chunk, D), jnp.float32),
            qbuf=pltpu.VMEM((2, chunk, D), jnp.float32),
            uqbuf=pltpu.VMEM((NCH, chunk, L), jnp.float32),
            ovbuf=pltpu.VMEM((chunk, D), jnp.float32),
            msbuf=pltpu.VMEM((qpw,), jnp.float32),
            sems=pltpu.SemaphoreType.DMA((2, 5)),
        ),
    )
    def sc_call(qf, kt, vals, stids, cents, tidsf, val_out, ms_out, bk_out,
                tid_v, bkt_v, ktbuf, vbuf, stbuf, cbuf, qbuf, uqbuf,
                ovbuf, msbuf, sems):
        wid = lax.axis_index("s") * nc + lax.axis_index("c")
        base = wid * qpw

        pltpu.sync_copy(tidsf.at[pl.ds(base, qpw)], tid_v)

        def bkt_body(i, _):
            t16 = tid_v[pl.ds(i * L, L)]
            bkt_v[pl.ds(i * L, L)] = t16 & jnp.int32(N_BUCKETS - 1)
            return 0

        lax.fori_loop(0, qpw // L, bkt_body, 0)
        pltpu.sync_copy(bkt_v, bk_out.at[pl.ds(base, qpw)])

        zero16 = jnp.zeros((L,), jnp.float32)
        lanes = lax.iota(jnp.int32, L)

        def fire(g, s):
            idx = bkt_v.at[pl.ds(g * chunk, chunk)]
            pltpu.async_copy(kt.at[idx], ktbuf.at[s], sems.at[s, 0])
            pltpu.async_copy(vals.at[idx], vbuf.at[s], sems.at[s, 1])
            pltpu.async_copy(stids.at[idx], stbuf.at[s], sems.at[s, 2])
            pltpu.async_copy(cents.at[idx], cbuf.at[s], sems.at[s, 3])
            pltpu.async_copy(qf.at[pl.ds(base + g * chunk, chunk)],
                             qbuf.at[s], sems.at[s, 4])

        def wait(g, s):
            idx = bkt_v.at[pl.ds(g * chunk, chunk)]
            pltpu.make_async_copy(kt.at[idx], ktbuf.at[s], sems.at[s, 0]).wait()
            pltpu.make_async_copy(vals.at[idx], vbuf.at[s], sems.at[s, 1]).wait()
            pltpu.make_async_copy(stids.at[idx], stbuf.at[s], sems.at[s, 2]).wait()
            pltpu.make_async_copy(cents.at[idx], cbuf.at[s], sems.at[s, 3]).wait()
            pltpu.make_async_copy(qf.at[pl.ds(base + g * chunk, chunk)],
                                  qbuf.at[s], sems.at[s, 4]).wait()

        def compute(g, s, lane0, tchunk, msv):
            row0 = g * chunk
            for qi in range(chunk):
                qs = [qbuf[s, qi, pl.ds(c * L, L)] for c in range(NCH)]
                nsq = zero16
                for q_c in qs:
                    nsq = nsq + q_c * q_c
                nsq = jnp.maximum(_sum_all(nsq, lanes), 1e-24)
                rq = _rsqrt16(nsq)
                ts = [qs[c] * rq + cbuf[s, qi, pl.ds(c * L, L)]
                      for c in range(NCH)]
                tsq = zero16
                for t_c in ts:
                    tsq = tsq + t_c * t_c
                tsq = jnp.maximum(_sum_all(tsq, lanes), 1e-24)
                rt = _rsqrt16(tsq)
                for c in range(NCH):
                    uqbuf[c, qi, :] = _round_bf16(ts[c] * rt)

            def score_body(c, carry):
                accs = list(carry)
                for qi in range(chunk):
                    uq_c = uqbuf[c, qi, :]
                    for j in range(L):
                        u = _bcast_lane(uq_c, j)
                        dd = c * L + j
                        k0, k1 = _unpack2(ktbuf[s, qi, pl.ds(dd * L, L)])
                        accs[2 * qi] = accs[2 * qi] + u * k0
                        accs[2 * qi + 1] = accs[2 * qi + 1] + u * k1
                return tuple(accs)

            sc_acc = lax.fori_loop(0, NCH, score_body, (zero16,) * (2 * chunk))

            ps = []
            for qi in range(chunk):
                s0, s1 = sc_acc[2 * qi], sc_acc[2 * qi + 1]
                tidv = _bcast_lane(tchunk, lane0 + qi)
                mf0 = 1.0 - jnp.minimum(
                    jnp.abs(stbuf[s, qi, pl.ds(0, L)] - tidv), 1
                ).astype(jnp.float32)
                mf1 = 1.0 - jnp.minimum(
                    jnp.abs(stbuf[s, qi, pl.ds(L, L)] - tidv), 1
                ).astype(jnp.float32)
                msum = _sum_all(mf0 + mf1, lanes)
                hasf = jnp.minimum(msum, 1.0)
                smax = _max_all(jnp.maximum(s0, s1), lanes)
                e0 = jnp.exp((s0 - smax) * (1.0 / TAU))
                e1 = jnp.exp((s1 - smax) * (1.0 / TAU))
                zinv = 1.0 / _sum_all(e0 + e1, lanes)
                hinv = 1.0 / (msum + 1e-9)
                p0 = _round_bf16(
                    hasf * (mf0 * hinv) + (1.0 - hasf) * (e0 * zinv))
                p1 = _round_bf16(
                    hasf * (mf1 * hinv) + (1.0 - hasf) * (e1 * zinv))
                ps.append((p0, p1))
                ms_q = hasf * 10.0 + (1.0 - hasf) * smax
                lm = jnp.minimum(
                    jnp.abs(lanes - (lane0 + qi)), 1).astype(jnp.float32)
                msv = msv * lm + ms_q * (1.0 - lm)

            for qi in range(chunk):
                p0, p1 = ps[qi]

                def val_body(so, accs):
                    sl = so & (L - 1)
                    svec = jnp.full((L,), so, jnp.int32)
                    hi = jnp.minimum(jnp.maximum(svec - (L - 1), 0),
                                     1).astype(jnp.float32)
                    b = (1.0 - hi) * _bcast_lane(p0, sl) \
                        + hi * _bcast_lane(p1, sl)
                    out = []
                    for c in range(NCH // 2):
                        va, vb = _unpack2(
                            vbuf[s, qi, pl.ds(so * (D // 2) + c * L, L)])
                        out.append(accs[2 * c] + b * va)
                        out.append(accs[2 * c + 1] + b * vb)
                    return tuple(out)

                accs = lax.fori_loop(0, SLOTS, val_body, (zero16,) * NCH,
                                     unroll=2)
                for c in range(NCH):
                    ovbuf[qi, pl.ds(c * L, L)] = accs[c]

            pltpu.sync_copy(ovbuf, val_out.at[pl.ds(base + row0, chunk)])
            return msv

        fire(0, 0)

        def pair_body(h, _):
            g0 = 2 * h
            tchunk = tid_v[pl.ds(h * L, L)]
            fire(g0 + 1, 1)
            wait(g0, 0)
            msv = compute(g0, 0, 0, tchunk, zero16)

            @pl.when(h + 1 < grids // 2)
            def _():
                fire(g0 + 2, 0)

            wait(g0 + 1, 1)
            msv = compute(g0 + 1, 1, chunk, tchunk, msv)
            msbuf[pl.ds(g0 * chunk, L)] = msv
            return 0

        lax.fori_loop(0, grids // 2, pair_body, 0)
        pltpu.sync_copy(msbuf, ms_out.at[pl.ds(base, qpw)])

    return sc_call


def kernel(query_emb, slot_values, slot_keys, tids, centroid_codebook,
           slot_tids):
    B, T, d = query_emb.shape
    nq = B * T
    nw = 32
    qpw = nq // nw

    qf = query_emb.reshape(nq, d)
    ktf = jnp.transpose(slot_keys[0].reshape(N_BUCKETS, SLOTS, d), (0, 2, 1))
    kt = lax.bitcast_convert_type(
        ktf.astype(jnp.bfloat16)
        .reshape(N_BUCKETS, d, 2, L)
        .transpose(0, 1, 3, 2)
        .reshape(N_BUCKETS, d * SLOTS // 2, 2), jnp.int32)
    vals = lax.bitcast_convert_type(
        slot_values.reshape(N_BUCKETS, SLOTS, d).astype(jnp.bfloat16)
        .reshape(N_BUCKETS, SLOTS, NCH // 2, 2, L)
        .transpose(0, 1, 2, 4, 3)
        .reshape(N_BUCKETS, SLOTS * d // 2, 2), jnp.int32)
    stids = jnp.pad(slot_tids[0].reshape(N_BUCKETS, SLOTS).astype(jnp.int32),
                    ((0, 0), (0, d - SLOTS)), constant_values=-1)
    cents = centroid_codebook
    tidsf = tids.reshape(nq).astype(jnp.int32)

    sc_call = _make_sc_call(nq, qpw, chunk=8)
    val, ms, bk = sc_call(qf, kt, vals, stids, cents, tidsf)
    return val.reshape(B, T, d), ms.reshape(B, T), bk.reshape(B, T)


# speedup vs baseline: 1.3764x; 1.0459x over previous
import functools

import jax
import jax.numpy as jnp
from jax import lax
from jax.experimental import pallas as pl
from jax.experimental.pallas import tpu as pltpu
from jax.experimental.pallas import tpu_sc as plsc

N_BUCKETS = 512
SLOTS = 32
D = 128
NCH = D // 16
TAU = 0.1
L = 16


def _rsqrt16(x):
    i = lax.bitcast_convert_type(x, jnp.int32)
    i = jnp.int32(0x5F3759DF) - (i >> 1)
    y = lax.bitcast_convert_type(i, jnp.float32)
    for _ in range(3):
        y = y * (1.5 - 0.5 * x * y * y)
    return y


def _bcast_lane(v, lane):
    idx = jnp.full((L,), lane, jnp.int32)
    return v.at[idx].get(mode="promise_in_bounds")


def _round_bf16(v):
    i = lax.bitcast_convert_type(v, jnp.int32)
    i = i + jnp.int32(0x7FFF) + ((i >> 16) & 1)
    i = i & jnp.int32(-65536)
    return lax.bitcast_convert_type(i, jnp.float32)


def _sum_all(v, lanes):
    for sh in (8, 4, 2, 1):
        idx = lanes ^ sh
        v = v + v.at[idx].get(mode="promise_in_bounds")
    return v


def _max_all(v, lanes):
    for sh in (8, 4, 2, 1):
        idx = lanes ^ sh
        v = jnp.maximum(v, v.at[idx].get(mode="promise_in_bounds"))
    return v


def _unpack2(w):
    a = lax.bitcast_convert_type(w << 16, jnp.float32)
    b = lax.bitcast_convert_type(w & jnp.int32(-65536), jnp.float32)
    return a, b


def _make_sc_call(num_queries, qpw, chunk):
    mesh = plsc.VectorSubcoreMesh(core_axis_name="c", subcore_axis_name="s",
                                  num_cores=2, num_subcores=16)
    nc = 2
    grids = qpw // chunk
    assert grids % 2 == 0 and chunk == 8

    @functools.partial(
        pl.kernel,
        out_type=(
            jax.ShapeDtypeStruct((num_queries, D), jnp.float32),
            jax.ShapeDtypeStruct((num_queries,), jnp.float32),
            jax.ShapeDtypeStruct((num_queries,), jnp.int32),
        ),
        mesh=mesh,
        scratch_types=dict(
            tid_v=pltpu.VMEM((qpw,), jnp.int32),
            bkt_v=pltpu.VMEM((qpw,), jnp.int32),
            ktbuf=pltpu.VMEM((2, chunk, D * SLOTS // 2), jnp.int32),
            vbuf=pltpu.VMEM((2, chunk, SLOTS * D // 2), jnp.int32),
            stbuf=pltpu.VMEM((2, chunk, D), jnp.int32),
            cbuf=pltpu.VMEM((2, chunk, D), jnp.float32),
            qbuf=pltpu.VMEM((2, chunk, D), jnp.float32),
            uqbuf=pltpu.VMEM((NCH, chunk, L), jnp.float32),
            ovbuf=pltpu.VMEM((2, chunk, D), jnp.float32),
            msbuf=pltpu.VMEM((qpw,), jnp.float32),
            sems=pltpu.SemaphoreType.DMA((2, 5)),
            osems=pltpu.SemaphoreType.DMA((2,)),
        ),
    )
    def sc_call(qf, kt, vals, stids, cents, tidsf, val_out, ms_out, bk_out,
                tid_v, bkt_v, ktbuf, vbuf, stbuf, cbuf, qbuf, uqbuf,
                ovbuf, msbuf, sems, osems):
        wid = lax.axis_index("s") * nc + lax.axis_index("c")
        base = wid * qpw

        pltpu.sync_copy(tidsf.at[pl.ds(base, qpw)], tid_v)

        def bkt_body(i, _):
            t16 = tid_v[pl.ds(i * L, L)]
            bkt_v[pl.ds(i * L, L)] = t16 & jnp.int32(N_BUCKETS - 1)
            return 0

        lax.fori_loop(0, qpw // L, bkt_body, 0)
        pltpu.sync_copy(bkt_v, bk_out.at[pl.ds(base, qpw)])

        zero16 = jnp.zeros((L,), jnp.float32)
        lanes = lax.iota(jnp.int32, L)

        def fire(g, s):
            idx = bkt_v.at[pl.ds(g * chunk, chunk)]
            pltpu.async_copy(kt.at[idx], ktbuf.at[s], sems.at[s, 0])
            pltpu.async_copy(vals.at[idx], vbuf.at[s], sems.at[s, 1])
            pltpu.async_copy(stids.at[idx], stbuf.at[s], sems.at[s, 2])
            pltpu.async_copy(cents.at[idx], cbuf.at[s], sems.at[s, 3])
            pltpu.async_copy(qf.at[pl.ds(base + g * chunk, chunk)],
                             qbuf.at[s], sems.at[s, 4])

        def wait(g, s):
            idx = bkt_v.at[pl.ds(g * chunk, chunk)]
            pltpu.make_async_copy(kt.at[idx], ktbuf.at[s], sems.at[s, 0]).wait()
            pltpu.make_async_copy(vals.at[idx], vbuf.at[s], sems.at[s, 1]).wait()
            pltpu.make_async_copy(stids.at[idx], stbuf.at[s], sems.at[s, 2]).wait()
            pltpu.make_async_copy(cents.at[idx], cbuf.at[s], sems.at[s, 3]).wait()
            pltpu.make_async_copy(qf.at[pl.ds(base + g * chunk, chunk)],
                                  qbuf.at[s], sems.at[s, 4]).wait()

        def compute(g, s, lane0, tchunk, msv):
            row0 = g * chunk
            for qi in range(chunk):
                qs = [qbuf[s, qi, pl.ds(c * L, L)] for c in range(NCH)]
                nsq = zero16
                for q_c in qs:
                    nsq = nsq + q_c * q_c
                nsq = jnp.maximum(_sum_all(nsq, lanes), 1e-24)
                rq = _rsqrt16(nsq)
                ts = [qs[c] * rq + cbuf[s, qi, pl.ds(c * L, L)]
                      for c in range(NCH)]
                tsq = zero16
                for t_c in ts:
                    tsq = tsq + t_c * t_c
                tsq = jnp.maximum(_sum_all(tsq, lanes), 1e-24)
                rt = _rsqrt16(tsq)
                for c in range(NCH):
                    uqbuf[c, qi, :] = _round_bf16(ts[c] * rt)

            def score_body(c, carry):
                accs = list(carry)
                for qi in range(chunk):
                    uq_c = uqbuf[c, qi, :]
                    for j in range(L):
                        u = _bcast_lane(uq_c, j)
                        dd = c * L + j
                        k0, k1 = _unpack2(ktbuf[s, qi, pl.ds(dd * L, L)])
                        accs[2 * qi] = accs[2 * qi] + u * k0
                        accs[2 * qi + 1] = accs[2 * qi + 1] + u * k1
                return tuple(accs)

            sc_acc = lax.fori_loop(0, NCH, score_body, (zero16,) * (2 * chunk))

            ps = []
            for qi in range(chunk):
                s0, s1 = sc_acc[2 * qi], sc_acc[2 * qi + 1]
                tidv = _bcast_lane(tchunk, lane0 + qi)
                mf0 = 1.0 - jnp.minimum(
                    jnp.abs(stbuf[s, qi, pl.ds(0, L)] - tidv), 1
                ).astype(jnp.float32)
                mf1 = 1.0 - jnp.minimum(
                    jnp.abs(stbuf[s, qi, pl.ds(L, L)] - tidv), 1
                ).astype(jnp.float32)
                msum = _sum_all(mf0 + mf1, lanes)
                hasf = jnp.minimum(msum, 1.0)
                smax = _max_all(jnp.maximum(s0, s1), lanes)
                e0 = jnp.exp((s0 - smax) * (1.0 / TAU))
                e1 = jnp.exp((s1 - smax) * (1.0 / TAU))
                zinv = 1.0 / _sum_all(e0 + e1, lanes)
                hinv = 1.0 / (msum + 1e-9)
                p0 = _round_bf16(
                    hasf * (mf0 * hinv) + (1.0 - hasf) * (e0 * zinv))
                p1 = _round_bf16(
                    hasf * (mf1 * hinv) + (1.0 - hasf) * (e1 * zinv))
                ps.append((p0, p1))
                ms_q = hasf * 10.0 + (1.0 - hasf) * smax
                lm = jnp.minimum(
                    jnp.abs(lanes - (lane0 + qi)), 1).astype(jnp.float32)
                msv = msv * lm + ms_q * (1.0 - lm)

            @pl.when(g >= 2)
            def _():
                pltpu.make_async_copy(
                    ovbuf.at[s],
                    val_out.at[pl.ds(base + (g - 2) * chunk, chunk)],
                    osems.at[s]).wait()

            for q0 in range(0, chunk, 2):
                pp = [ps[q0], ps[q0 + 1]]

                def val_body(so, accs):
                    sl = so & (L - 1)
                    svec = jnp.full((L,), so, jnp.int32)
                    hi = jnp.minimum(jnp.maximum(svec - (L - 1), 0),
                                     1).astype(jnp.float32)
                    hi1 = 1.0 - hi
                    out = list(accs)
                    for k in range(2):
                        p0, p1 = pp[k]
                        b = hi1 * _bcast_lane(p0, sl) + hi * _bcast_lane(p1, sl)
                        for c in range(NCH // 2):
                            va, vb = _unpack2(
                                vbuf[s, q0 + k,
                                     pl.ds(so * (D // 2) + c * L, L)])
                            out[k * NCH + 2 * c] = \
                                out[k * NCH + 2 * c] + b * va
                            out[k * NCH + 2 * c + 1] = \
                                out[k * NCH + 2 * c + 1] + b * vb
                    return tuple(out)

                accs = lax.fori_loop(0, SLOTS, val_body,
                                     (zero16,) * (2 * NCH))
                for k in range(2):
                    for c in range(NCH):
                        ovbuf[s, q0 + k, pl.ds(c * L, L)] = accs[k * NCH + c]

            pltpu.async_copy(ovbuf.at[s],
                             val_out.at[pl.ds(base + row0, chunk)],
                             osems.at[s])
            return msv

        fire(0, 0)

        def pair_body(h, _):
            g0 = 2 * h
            tchunk = tid_v[pl.ds(h * L, L)]
            fire(g0 + 1, 1)
            wait(g0, 0)
            msv = compute(g0, 0, 0, tchunk, zero16)

            @pl.when(h + 1 < grids // 2)
            def _():
                fire(g0 + 2, 0)

            wait(g0 + 1, 1)
            msv = compute(g0 + 1, 1, chunk, tchunk, msv)
            msbuf[pl.ds(g0 * chunk, L)] = msv
            return 0

        lax.fori_loop(0, grids // 2, pair_body, 0)
        for s_ in range(2):
            pltpu.make_async_copy(
                ovbuf.at[s_],
                val_out.at[pl.ds(base + (grids - 2 + s_) * chunk, chunk)],
                osems.at[s_]).wait()
        pltpu.sync_copy(msbuf, ms_out.at[pl.ds(base, qpw)])

    return sc_call


def kernel(query_emb, slot_values, slot_keys, tids, centroid_codebook,
           slot_tids):
    B, T, d = query_emb.shape
    nq = B * T
    nw = 32
    qpw = nq // nw

    qf = query_emb.reshape(nq, d)
    ktf = jnp.transpose(slot_keys[0].reshape(N_BUCKETS, SLOTS, d), (0, 2, 1))
    kt = lax.bitcast_convert_type(
        ktf.astype(jnp.bfloat16)
        .reshape(N_BUCKETS, d, 2, L)
        .transpose(0, 1, 3, 2)
        .reshape(N_BUCKETS, d * SLOTS // 2, 2), jnp.int32)
    vals = lax.bitcast_convert_type(
        slot_values.reshape(N_BUCKETS, SLOTS, d).astype(jnp.bfloat16)
        .reshape(N_BUCKETS, SLOTS, NCH // 2, 2, L)
        .transpose(0, 1, 2, 4, 3)
        .reshape(N_BUCKETS, SLOTS * d // 2, 2), jnp.int32)
    stids = jnp.pad(slot_tids[0].reshape(N_BUCKETS, SLOTS).astype(jnp.int32),
                    ((0, 0), (0, d - SLOTS)), constant_values=-1)
    cents = centroid_codebook
    tidsf = tids.reshape(nq).astype(jnp.int32)

    sc_call = _make_sc_call(nq, qpw, chunk=8)
    val, ms, bk = sc_call(qf, kt, vals, stids, cents, tidsf)
    return val.reshape(B, T, d), ms.reshape(B, T), bk.reshape(B, T)


# speedup vs baseline: 1.3813x; 1.0036x over previous
import functools

import jax
import jax.numpy as jnp
from jax import lax
from jax.experimental import pallas as pl
from jax.experimental.pallas import tpu as pltpu
from jax.experimental.pallas import tpu_sc as plsc

N_BUCKETS = 512
SLOTS = 32
D = 128
NCH = D // 16
TAU = 0.1
L = 16


def _rsqrt16(x):
    i = lax.bitcast_convert_type(x, jnp.int32)
    i = jnp.int32(0x5F3759DF) - (i >> 1)
    y = lax.bitcast_convert_type(i, jnp.float32)
    for _ in range(3):
        y = y * (1.5 - 0.5 * x * y * y)
    return y


def _bcast_lane(v, lane):
    idx = jnp.full((L,), lane, jnp.int32)
    return v.at[idx].get(mode="promise_in_bounds")


def _round_bf16(v):
    i = lax.bitcast_convert_type(v, jnp.int32)
    i = i + jnp.int32(0x7FFF) + ((i >> 16) & 1)
    i = i & jnp.int32(-65536)
    return lax.bitcast_convert_type(i, jnp.float32)


def _sum_all(v, lanes):
    for sh in (8, 4, 2, 1):
        idx = lanes ^ sh
        v = v + v.at[idx].get(mode="promise_in_bounds")
    return v


def _max_all(v, lanes):
    for sh in (8, 4, 2, 1):
        idx = lanes ^ sh
        v = jnp.maximum(v, v.at[idx].get(mode="promise_in_bounds"))
    return v


def _unpack2(w):
    a = lax.bitcast_convert_type(w << 16, jnp.float32)
    b = lax.bitcast_convert_type(w & jnp.int32(-65536), jnp.float32)
    return a, b


def _make_sc_call(num_queries, qpw, chunk):
    mesh = plsc.VectorSubcoreMesh(core_axis_name="c", subcore_axis_name="s",
                                  num_cores=2, num_subcores=16)
    nc = 2
    grids = qpw // chunk
    assert grids % 2 == 0 and chunk == 8

    @functools.partial(
        pl.kernel,
        out_type=(
            jax.ShapeDtypeStruct((num_queries, D), jnp.float32),
            jax.ShapeDtypeStruct((num_queries,), jnp.float32),
            jax.ShapeDtypeStruct((num_queries,), jnp.int32),
        ),
        mesh=mesh,
        scratch_types=dict(
            tid_v=pltpu.VMEM((qpw,), jnp.int32),
            bkt_v=pltpu.VMEM((qpw,), jnp.int32),
            ktbuf=pltpu.VMEM((2, chunk, D * SLOTS // 2), jnp.int32),
            vbuf=pltpu.VMEM((2, chunk, SLOTS * D // 2), jnp.int32),
            stbuf=pltpu.VMEM((2, chunk, D), jnp.int32),
            cbuf=pltpu.VMEM((2, chunk, D), jnp.float32),
            qbuf=pltpu.VMEM((2, chunk, D), jnp.float32),
            uqbuf=pltpu.VMEM((NCH, chunk, L), jnp.float32),
            ovbuf=pltpu.VMEM((2, chunk, D), jnp.float32),
            msbuf=pltpu.VMEM((qpw,), jnp.float32),
            sems=pltpu.SemaphoreType.DMA((2, 5)),
            osems=pltpu.SemaphoreType.DMA((2,)),
        ),
    )
    def sc_call(qf, kt, vals, stids, cents, tidsf, val_out, ms_out, bk_out,
                tid_v, bkt_v, ktbuf, vbuf, stbuf, cbuf, qbuf, uqbuf,
                ovbuf, msbuf, sems, osems):
        wid = lax.axis_index("s") * nc + lax.axis_index("c")
        base = wid * qpw

        pltpu.sync_copy(tidsf.at[pl.ds(base, qpw)], tid_v)

        def bkt_body(i, _):
            t16 = tid_v[pl.ds(i * L, L)]
            bkt_v[pl.ds(i * L, L)] = t16 & jnp.int32(N_BUCKETS - 1)
            return 0

        lax.fori_loop(0, qpw // L, bkt_body, 0)
        pltpu.sync_copy(bkt_v, bk_out.at[pl.ds(base, qpw)])

        zero16 = jnp.zeros((L,), jnp.float32)
        lanes = lax.iota(jnp.int32, L)

        def fire(g, s):
            idx = bkt_v.at[pl.ds(g * chunk, chunk)]
            pltpu.async_copy(kt.at[idx], ktbuf.at[s], sems.at[s, 0])
            pltpu.async_copy(vals.at[idx], vbuf.at[s], sems.at[s, 1])
            pltpu.async_copy(stids.at[idx], stbuf.at[s], sems.at[s, 2])
            pltpu.async_copy(cents.at[idx], cbuf.at[s], sems.at[s, 3])
            pltpu.async_copy(qf.at[pl.ds(base + g * chunk, chunk)],
                             qbuf.at[s], sems.at[s, 4])

        def wait(g, s):
            idx = bkt_v.at[pl.ds(g * chunk, chunk)]
            pltpu.make_async_copy(kt.at[idx], ktbuf.at[s], sems.at[s, 0]).wait()
            pltpu.make_async_copy(vals.at[idx], vbuf.at[s], sems.at[s, 1]).wait()
            pltpu.make_async_copy(stids.at[idx], stbuf.at[s], sems.at[s, 2]).wait()
            pltpu.make_async_copy(cents.at[idx], cbuf.at[s], sems.at[s, 3]).wait()
            pltpu.make_async_copy(qf.at[pl.ds(base + g * chunk, chunk)],
                                  qbuf.at[s], sems.at[s, 4]).wait()

        def compute(g, s, lane0, tchunk, msv):
            row0 = g * chunk
            for qi in range(chunk):
                qs = [qbuf[s, qi, pl.ds(c * L, L)] for c in range(NCH)]
                nsq = zero16
                for q_c in qs:
                    nsq = nsq + q_c * q_c
                nsq = jnp.maximum(_sum_all(nsq, lanes), 1e-24)
                rq = _rsqrt16(nsq)
                ts = [qs[c] * rq + cbuf[s, qi, pl.ds(c * L, L)]
                      for c in range(NCH)]
                tsq = zero16
                for t_c in ts:
                    tsq = tsq + t_c * t_c
                tsq = jnp.maximum(_sum_all(tsq, lanes), 1e-24)
                rt = _rsqrt16(tsq)
                for c in range(NCH):
                    uqbuf[c, qi, :] = _round_bf16(ts[c] * rt)

            def score_body(c, carry):
                accs = list(carry)
                for qi in range(chunk):
                    uq_c = uqbuf[c, qi, :]
                    for j in range(L):
                        u = _bcast_lane(uq_c, j)
                        dd = c * L + j
                        k0, k1 = _unpack2(ktbuf[s, qi, pl.ds(dd * L, L)])
                        accs[2 * qi] = accs[2 * qi] + u * k0
                        accs[2 * qi + 1] = accs[2 * qi + 1] + u * k1
                return tuple(accs)

            sc_acc = lax.fori_loop(0, NCH, score_body, (zero16,) * (2 * chunk))

            ps = []
            for qi in range(chunk):
                s0, s1 = sc_acc[2 * qi], sc_acc[2 * qi + 1]
                tidv = _bcast_lane(tchunk, lane0 + qi)
                mf0 = 1.0 - jnp.minimum(
                    jnp.abs(stbuf[s, qi, pl.ds(0, L)] - tidv), 1
                ).astype(jnp.float32)
                mf1 = 1.0 - jnp.minimum(
                    jnp.abs(stbuf[s, qi, pl.ds(L, L)] - tidv), 1
                ).astype(jnp.float32)
                msum = _sum_all(mf0 + mf1, lanes)
                hasf = jnp.minimum(msum, 1.0)
                smax = _max_all(jnp.maximum(s0, s1), lanes)
                e0 = jnp.exp((s0 - smax) * (1.0 / TAU))
                e1 = jnp.exp((s1 - smax) * (1.0 / TAU))
                zinv = 1.0 / _sum_all(e0 + e1, lanes)
                hinv = 1.0 / (msum + 1e-9)
                p0 = _round_bf16(
                    hasf * (mf0 * hinv) + (1.0 - hasf) * (e0 * zinv))
                p1 = _round_bf16(
                    hasf * (mf1 * hinv) + (1.0 - hasf) * (e1 * zinv))
                ps.append((p0, p1))
                ms_q = hasf * 10.0 + (1.0 - hasf) * smax
                lm = jnp.minimum(
                    jnp.abs(lanes - (lane0 + qi)), 1).astype(jnp.float32)
                msv = msv * lm + ms_q * (1.0 - lm)

            @pl.when(g >= 2)
            def _():
                pltpu.make_async_copy(
                    ovbuf.at[s],
                    val_out.at[pl.ds(base + (g - 2) * chunk, chunk)],
                    osems.at[s]).wait()

            for q0 in range(0, chunk, 2):
                pp = [ps[q0], ps[q0 + 1]]

                def make_val_body(half):
                    def val_body(sl, accs):
                        so = sl + half * L
                        out = list(accs)
                        for k in range(2):
                            b = _bcast_lane(pp[k][half], sl)
                            for c in range(NCH // 2):
                                va, vb = _unpack2(
                                    vbuf[s, q0 + k,
                                         pl.ds(so * (D // 2) + c * L, L)])
                                out[k * NCH + 2 * c] = \
                                    out[k * NCH + 2 * c] + b * va
                                out[k * NCH + 2 * c + 1] = \
                                    out[k * NCH + 2 * c + 1] + b * vb
                        return tuple(out)
                    return val_body

                accs = lax.fori_loop(0, L, make_val_body(0),
                                     (zero16,) * (2 * NCH))
                accs = lax.fori_loop(0, L, make_val_body(1), accs)
                for k in range(2):
                    for c in range(NCH):
                        ovbuf[s, q0 + k, pl.ds(c * L, L)] = accs[k * NCH + c]

            pltpu.async_copy(ovbuf.at[s],
                             val_out.at[pl.ds(base + row0, chunk)],
                             osems.at[s])
            return msv

        fire(0, 0)

        def pair_body(h, _):
            g0 = 2 * h
            tchunk = tid_v[pl.ds(h * L, L)]
            fire(g0 + 1, 1)
            wait(g0, 0)
            msv = compute(g0, 0, 0, tchunk, zero16)

            @pl.when(h + 1 < grids // 2)
            def _():
                fire(g0 + 2, 0)

            wait(g0 + 1, 1)
            msv = compute(g0 + 1, 1, chunk, tchunk, msv)
            msbuf[pl.ds(g0 * chunk, L)] = msv
            return 0

        lax.fori_loop(0, grids // 2, pair_body, 0)
        for s_ in range(2):
            pltpu.make_async_copy(
                ovbuf.at[s_],
                val_out.at[pl.ds(base + (grids - 2 + s_) * chunk, chunk)],
                osems.at[s_]).wait()
        pltpu.sync_copy(msbuf, ms_out.at[pl.ds(base, qpw)])

    return sc_call


def kernel(query_emb, slot_values, slot_keys, tids, centroid_codebook,
           slot_tids):
    B, T, d = query_emb.shape
    nq = B * T
    nw = 32
    qpw = nq // nw

    qf = query_emb.reshape(nq, d)
    ktf = jnp.transpose(slot_keys[0].reshape(N_BUCKETS, SLOTS, d), (0, 2, 1))
    kt = lax.bitcast_convert_type(
        ktf.astype(jnp.bfloat16)
        .reshape(N_BUCKETS, d, 2, L)
        .transpose(0, 1, 3, 2)
        .reshape(N_BUCKETS, d * SLOTS // 2, 2), jnp.int32)
    vals = lax.bitcast_convert_type(
        slot_values.reshape(N_BUCKETS, SLOTS, d).astype(jnp.bfloat16)
        .reshape(N_BUCKETS, SLOTS, NCH // 2, 2, L)
        .transpose(0, 1, 2, 4, 3)
        .reshape(N_BUCKETS, SLOTS * d // 2, 2), jnp.int32)
    stids = jnp.pad(slot_tids[0].reshape(N_BUCKETS, SLOTS).astype(jnp.int32),
                    ((0, 0), (0, d - SLOTS)), constant_values=-1)
    cents = centroid_codebook
    tidsf = tids.reshape(nq).astype(jnp.int32)

    sc_call = _make_sc_call(nq, qpw, chunk=8)
    val, ms, bk = sc_call(qf, kt, vals, stids, cents, tidsf)
    return val.reshape(B, T, d), ms.reshape(B, T), bk.reshape(B, T)
